# baseline (speedup 1.0000x reference)
import functools

import jax
import jax.numpy as jnp
from jax import lax
from jax.experimental import pallas as pl
from jax.experimental.pallas import tpu as pltpu
from jax.experimental.pallas import tpu_sc as plsc

FP = 64
B = 256
L = 256
K = 6
AF = 39
MPA = 8
MPC = 8

NC = 2
NS = 16
NW = NC * NS
MPT = B // NW

_NEG = -9e8


def _lk(x):
    return jax.nn.leaky_relu(x, 0.2)


def _stage_a(x_ref, wa_ref, ba_ref, alwa_ref, alwb_ref, alb_ref, atw_ref, atb_ref,
             af_ref, nft_ref, g_ref, asc_ref):
    f32 = jnp.float32
    for m in range(MPA):
        x = x_ref[m]
        af = _lk(jnp.dot(x, wa_ref[...], preferred_element_type=f32) + ba_ref[...])
        act = _lk(af)
        af_ref[m] = af
        nft_ref[m] = jnp.dot(act, atw_ref[...], preferred_element_type=f32) + atb_ref[...]
        g_ref[m] = jnp.dot(act, alwb_ref[...], preferred_element_type=f32)
        asc_ref[m] = jnp.dot(af, alwa_ref[...], preferred_element_type=f32) + alb_ref[...]


def _stage_b_body(nft_hbm, g_hbm, asc_hbm, idx_hbm,
                  ctx_hbm,
                  nft_v, g_v, asc_v, idx_v, ctx_v):
    wid = lax.axis_index("s") * NC + lax.axis_index("c")
    lane = lax.broadcasted_iota(jnp.int32, (16,), 0)

    def do_mol(j, carry):
        b = wid * MPT + j
        pltpu.sync_copy(nft_hbm.at[pl.ds(b * (L * FP), L * FP)], nft_v)
        pltpu.sync_copy(g_hbm.at[pl.ds(b * L, L)], g_v)
        pltpu.sync_copy(asc_hbm.at[pl.ds(b * L, L)], asc_v)
        pltpu.sync_copy(idx_hbm.at[pl.ds(b * (K * L), K * L)], idx_v)

        def group(a, carry2):
            base = a * 16
            asc = asc_v[pl.ds(base, 16)]
            idxs = [idx_v[pl.ds(k * L + base, 16)] for k in range(K)]
            gvs = [plsc.load_gather(g_v, [idxs[k]]) for k in range(K)]
            scs = []
            for k in range(K):
                t = asc + gvs[k]
                t = jnp.maximum(t, 0.2 * t)
                scs.append(jnp.where(idxs[k] == L - 1, t + _NEG, t))
            m = scs[0]
            for k in range(1, K):
                m = jnp.maximum(m, scs[k])
            es = [jnp.exp(scs[k] - m) for k in range(K)]
            s = es[0]
            for k in range(1, K):
                s = s + es[k]
            aws = [jnp.where(idxs[k] == L - 1, 0.0, es[k] / s) for k in range(K)]
            cols = [idxs[k] * FP for k in range(K)]
            st0 = (lane + base) * FP
            for f in range(FP):
                acc = aws[0] * plsc.load_gather(nft_v, [cols[0] + f])
                for k in range(1, K):
                    acc = acc + aws[k] * plsc.load_gather(nft_v, [cols[k] + f])
                plsc.store_scatter(ctx_v, [st0 + f], acc)
            return carry2

        lax.fori_loop(0, L // 16, group, 0)
        pltpu.sync_copy(ctx_v, ctx_hbm.at[pl.ds(b * (L * FP), L * FP)])
        return carry

    lax.fori_loop(0, MPT, do_mol, 0)


def _stage_b(nft_f, g_f, asc_f, idx_f):
    sc_call = functools.partial(
        pl.kernel,
        out_type=jax.ShapeDtypeStruct((B * L * FP,), jnp.float32),
        mesh=plsc.VectorSubcoreMesh(core_axis_name="c", subcore_axis_name="s",
                                    num_cores=NC, num_subcores=NS),
        scratch_types=[
            pltpu.VMEM((L * FP,), jnp.float32),
            pltpu.VMEM((L,), jnp.float32),
            pltpu.VMEM((L,), jnp.float32),
            pltpu.VMEM((K * L,), jnp.int32),
            pltpu.VMEM((L * FP,), jnp.float32),
        ],
        compiler_params=pltpu.CompilerParams(needs_layout_passes=False),
    )(_stage_b_body)
    return sc_call(nft_f, g_f, asc_f, idx_f)


def _stage_c(ctx_ref, af_ref, am_ref,
             gk_ref, grk_ref, gbi_ref, gbr_ref,
             mola_ref, molbw_ref, molb_ref,
             maw_ref, mab_ref,
             mgk_ref, mgrk_ref, mgbi_ref, mgbr_ref,
             l1w_ref, l1b_ref, l2w_ref, l2b_ref,
             ow_ref, ob_ref,
             out_ref):
    f32 = jnp.float32

    def atom_stage(m):
        af = af_ref[m]
        act = _lk(af)
        am = am_ref[m]
        ctx = ctx_ref[m]

        mg = jnp.dot(ctx, gk_ref[...], preferred_element_type=f32) + gbi_ref[...]
        hg = jnp.dot(af, grk_ref[...], preferred_element_type=f32) + gbr_ref[...]
        z = jax.nn.sigmoid(mg[:, :FP] + hg[:, :FP])
        r = jax.nn.sigmoid(mg[:, FP:2 * FP] + hg[:, FP:2 * FP])
        hh = jnp.tanh(mg[:, 2 * FP:] + r * hg[:, 2 * FP:])
        h = z * af + (1.0 - z) * hh

        mol_m = jnp.sum(h * am, axis=0, keepdims=True)
        aft = jnp.dot(act, maw_ref[...], preferred_element_type=f32) + mab_ref[...]
        q = jnp.dot(act, molbw_ref[...], preferred_element_type=f32)
        mmask = jnp.where(am == 0.0, _NEG, 0.0)
        return mol_m, aft, q, mmask, am

    per_mol = [atom_stage(m) for m in range(MPC)]
    mol = jnp.concatenate([pm[0] for pm in per_mol], axis=0)

    for _ in range(2):
        actm = _lk(mol)
        psc = jnp.dot(actm, mola_ref[...], preferred_element_type=f32)
        mcs = []
        for m in range(MPC):
            _, aft, q, mmask, am = per_mol[m]
            ms = _lk(psc[m:m + 1, 0:1] + q + molb_ref[...]) + mmask
            mmax = jnp.max(ms, axis=0, keepdims=True)
            me = jnp.exp(ms - mmax)
            mw = me / jnp.sum(me, axis=0, keepdims=True) * am
            mcs.append(jnp.sum(mw * aft, axis=0, keepdims=True))
        mcs_c = jnp.concatenate(mcs, axis=0)
        mc = jnp.where(mcs_c > 0, mcs_c, jnp.exp(jnp.minimum(mcs_c, 0.0)) - 1.0)
        a1 = jnp.dot(mc, mgk_ref[...], preferred_element_type=f32) + mgbi_ref[...]
        a2 = jnp.dot(mol, mgrk_ref[...], preferred_element_type=f32) + mgbr_ref[...]
        z2 = jax.nn.sigmoid(a1[:, :FP] + a2[:, :FP])
        r2_ = jax.nn.sigmoid(a1[:, FP:2 * FP] + a2[:, FP:2 * FP])
        hh2 = jnp.tanh(a1[:, 2 * FP:] + r2_ * a2[:, 2 * FP:])
        mol = z2 * mol + (1.0 - z2) * hh2

    r1 = _lk(jnp.dot(mol, l1w_ref[...], preferred_element_type=f32) + l1b_ref[...])
    r2 = _lk(jnp.dot(r1, l2w_ref[...], preferred_element_type=f32) + l2b_ref[...])
    o = jnp.dot(r2, ow_ref[...], preferred_element_type=f32) + ob_ref[...]
    out_ref[...] = o[:, :, None]


def kernel(atom_list, bond_list, atom_degree_list, bond_degree_list, atom_mask, params):
    p = params
    f32 = jnp.float32
    adl = atom_degree_list.astype(jnp.int32)
    adl_t = jnp.transpose(adl, (0, 2, 1)).reshape(B * K * L)
    am3 = atom_mask[..., None].astype(f32)
    alw = p['align_w_2']
    molw = p['mol_align_w']

    def r2(v):
        return v.reshape(1, -1).astype(f32)

    mol_spec_a = lambda shape: pl.BlockSpec(shape, lambda b: (b, 0, 0))
    par_spec = lambda shape: pl.BlockSpec(shape, lambda b: (0, 0))

    a_ops = [atom_list, p['atom_fc_w'], r2(p['atom_fc_b']),
             alw[:FP], alw[FP:], r2(p['align_b_2']),
             p['attend_w_2'], r2(p['attend_b_2'])]
    a_specs = [mol_spec_a((MPA, L, AF))] + [par_spec(op.shape) for op in a_ops[1:]]
    af, nft, g3, asc3 = pl.pallas_call(
        _stage_a,
        grid=(B // MPA,),
        in_specs=a_specs,
        out_specs=(
            pl.BlockSpec((MPA, L, FP), lambda b: (b, 0, 0)),
            pl.BlockSpec((MPA, L, FP), lambda b: (b, 0, 0)),
            pl.BlockSpec((MPA, L, 1), lambda b: (b, 0, 0)),
            pl.BlockSpec((MPA, L, 1), lambda b: (b, 0, 0)),
        ),
        out_shape=(
            jax.ShapeDtypeStruct((B, L, FP), f32),
            jax.ShapeDtypeStruct((B, L, FP), f32),
            jax.ShapeDtypeStruct((B, L, 1), f32),
            jax.ShapeDtypeStruct((B, L, 1), f32),
        ),
    )(*a_ops)

    ctx_flat = _stage_b(
        nft.reshape(B * L * FP),
        g3.reshape(B * L),
        asc3.reshape(B * L),
        adl_t,
    )
    ctx = ctx_flat.reshape(B, L, FP)

    c_ops = [
        ctx, af, am3,
        p['gru_k_2'], p['gru_rk_2'], r2(p['gru_bi_2']), r2(p['gru_br_2']),
        molw[:FP], molw[FP:], r2(p['mol_align_b']),
        p['mol_attend_w'], r2(p['mol_attend_b']),
        p['mol_gru_k'], p['mol_gru_rk'], r2(p['mol_gru_bi']), r2(p['mol_gru_br']),
        p['lin1_w'], r2(p['lin1_b']), p['lin2_w'], r2(p['lin2_b']),
        p['out_w'], r2(p['out_b']),
    ]
    c_specs = [mol_spec_a((MPC, L, FP)), mol_spec_a((MPC, L, FP)),
               mol_spec_a((MPC, L, 1))]
    c_specs += [par_spec(op.shape) for op in c_ops[3:]]
    out = pl.pallas_call(
        _stage_c,
        grid=(B // MPC,),
        in_specs=c_specs,
        out_specs=pl.BlockSpec((MPC, 1, 1), lambda b: (b, 0, 0)),
        out_shape=jax.ShapeDtypeStruct((B, 1, 1), f32),
    )(*c_ops)
    return out.reshape(B, 1)


# speedup vs baseline: 1.9294x; 1.9294x over previous
import functools

import jax
import jax.numpy as jnp
from jax import lax
from jax.experimental import pallas as pl
from jax.experimental.pallas import tpu as pltpu
from jax.experimental.pallas import tpu_sc as plsc

FP = 64
B = 256
L = 256
K = 6
AF = 39
MPA = 8
MPC = 8

NC = 2
NS = 16
NW = NC * NS
MPT = B // NW

_NEG = -9e8


def _lk(x):
    return jax.nn.leaky_relu(x, 0.2)


def _stage_a(x_ref, wa_ref, ba_ref, alwa_ref, alwb_ref, alb_ref, atw_ref, atb_ref,
             af_ref, nft_ref, g_ref, asc_ref):
    f32 = jnp.float32
    for m in range(MPA):
        x = x_ref[m]
        af = _lk(jnp.dot(x, wa_ref[...], preferred_element_type=f32) + ba_ref[...])
        act = _lk(af)
        af_ref[m] = af
        nft = jnp.dot(act, atw_ref[...], preferred_element_type=f32) + atb_ref[...]
        nft_ref[m] = jnp.transpose(nft)
        g_ref[m] = jnp.dot(act, alwb_ref[...], preferred_element_type=f32)
        asc_ref[m] = jnp.dot(af, alwa_ref[...], preferred_element_type=f32) + alb_ref[...]


def _stage_b_body(nft_hbm, g_hbm, asc_hbm, idx_hbm,
                  ctx_hbm,
                  nft_v, g_v, asc_v, idx_v, ctx_v):
    wid = lax.axis_index("s") * NC + lax.axis_index("c")

    def do_mol(j, carry):
        b = wid * MPT + j
        pltpu.sync_copy(nft_hbm.at[pl.ds(b * (L * FP), L * FP)], nft_v)
        pltpu.sync_copy(g_hbm.at[pl.ds(b * L, L)], g_v)
        pltpu.sync_copy(asc_hbm.at[pl.ds(b * L, L)], asc_v)
        pltpu.sync_copy(idx_hbm.at[pl.ds(b * (K * L), K * L)], idx_v)

        def group(a, carry2):
            base = a * 16
            asc = asc_v[pl.ds(base, 16)]
            idxs = [idx_v[pl.ds(k * L + base, 16)] for k in range(K)]
            gvs = [plsc.load_gather(g_v, [idxs[k]]) for k in range(K)]
            scs = []
            for k in range(K):
                t = asc + gvs[k]
                t = jnp.maximum(t, 0.2 * t)
                scs.append(jnp.where(idxs[k] == L - 1, t + _NEG, t))
            m = scs[0]
            for k in range(1, K):
                m = jnp.maximum(m, scs[k])
            es = [jnp.exp(scs[k] - m) for k in range(K)]
            s = es[0]
            for k in range(1, K):
                s = s + es[k]
            aws = [jnp.where(idxs[k] == L - 1, 0.0, es[k] / s) for k in range(K)]
            for f in range(FP):
                acc = aws[0] * plsc.load_gather(nft_v, [idxs[0] + f * L])
                for k in range(1, K):
                    acc = acc + aws[k] * plsc.load_gather(nft_v, [idxs[k] + f * L])
                ctx_v[pl.ds(f * L + base, 16)] = acc
            return carry2

        lax.fori_loop(0, L // 16, group, 0)
        pltpu.sync_copy(ctx_v, ctx_hbm.at[pl.ds(b * (L * FP), L * FP)])
        return carry

    lax.fori_loop(0, MPT, do_mol, 0)


def _stage_b(nft_f, g_f, asc_f, idx_f):
    sc_call = functools.partial(
        pl.kernel,
        out_type=jax.ShapeDtypeStruct((B * L * FP,), jnp.float32),
        mesh=plsc.VectorSubcoreMesh(core_axis_name="c", subcore_axis_name="s",
                                    num_cores=NC, num_subcores=NS),
        scratch_types=[
            pltpu.VMEM((L * FP,), jnp.float32),
            pltpu.VMEM((L,), jnp.float32),
            pltpu.VMEM((L,), jnp.float32),
            pltpu.VMEM((K * L,), jnp.int32),
            pltpu.VMEM((L * FP,), jnp.float32),
        ],
        compiler_params=pltpu.CompilerParams(needs_layout_passes=False),
    )(_stage_b_body)
    return sc_call(nft_f, g_f, asc_f, idx_f)


def _stage_c(ctx_ref, af_ref, am_ref,
             gk_ref, grk_ref, gbi_ref, gbr_ref,
             mola_ref, molbw_ref, molb_ref,
             maw_ref, mab_ref,
             mgk_ref, mgrk_ref, mgbi_ref, mgbr_ref,
             l1w_ref, l1b_ref, l2w_ref, l2b_ref,
             ow_ref, ob_ref,
             out_ref):
    f32 = jnp.float32

    def atom_stage(m):
        af = af_ref[m]
        act = _lk(af)
        am = am_ref[m]
        ctx = jnp.transpose(ctx_ref[m])

        mg = jnp.dot(ctx, gk_ref[...], preferred_element_type=f32) + gbi_ref[...]
        hg = jnp.dot(af, grk_ref[...], preferred_element_type=f32) + gbr_ref[...]
        z = jax.nn.sigmoid(mg[:, :FP] + hg[:, :FP])
        r = jax.nn.sigmoid(mg[:, FP:2 * FP] + hg[:, FP:2 * FP])
        hh = jnp.tanh(mg[:, 2 * FP:] + r * hg[:, 2 * FP:])
        h = z * af + (1.0 - z) * hh

        mol_m = jnp.sum(h * am, axis=0, keepdims=True)
        aft = jnp.dot(act, maw_ref[...], preferred_element_type=f32) + mab_ref[...]
        q = jnp.dot(act, molbw_ref[...], preferred_element_type=f32)
        mmask = jnp.where(am == 0.0, _NEG, 0.0)
        return mol_m, aft, q, mmask, am

    per_mol = [atom_stage(m) for m in range(MPC)]
    mol = jnp.concatenate([pm[0] for pm in per_mol], axis=0)

    for _ in range(2):
        actm = _lk(mol)
        psc = jnp.dot(actm, mola_ref[...], preferred_element_type=f32)
        mcs = []
        for m in range(MPC):
            _, aft, q, mmask, am = per_mol[m]
            ms = _lk(psc[m:m + 1, 0:1] + q + molb_ref[...]) + mmask
            mmax = jnp.max(ms, axis=0, keepdims=True)
            me = jnp.exp(ms - mmax)
            mw = me / jnp.sum(me, axis=0, keepdims=True) * am
            mcs.append(jnp.sum(mw * aft, axis=0, keepdims=True))
        mcs_c = jnp.concatenate(mcs, axis=0)
        mc = jnp.where(mcs_c > 0, mcs_c, jnp.exp(jnp.minimum(mcs_c, 0.0)) - 1.0)
        a1 = jnp.dot(mc, mgk_ref[...], preferred_element_type=f32) + mgbi_ref[...]
        a2 = jnp.dot(mol, mgrk_ref[...], preferred_element_type=f32) + mgbr_ref[...]
        z2 = jax.nn.sigmoid(a1[:, :FP] + a2[:, :FP])
        r2_ = jax.nn.sigmoid(a1[:, FP:2 * FP] + a2[:, FP:2 * FP])
        hh2 = jnp.tanh(a1[:, 2 * FP:] + r2_ * a2[:, 2 * FP:])
        mol = z2 * mol + (1.0 - z2) * hh2

    r1 = _lk(jnp.dot(mol, l1w_ref[...], preferred_element_type=f32) + l1b_ref[...])
    r2 = _lk(jnp.dot(r1, l2w_ref[...], preferred_element_type=f32) + l2b_ref[...])
    o = jnp.dot(r2, ow_ref[...], preferred_element_type=f32) + ob_ref[...]
    out_ref[...] = o[:, :, None]


def kernel(atom_list, bond_list, atom_degree_list, bond_degree_list, atom_mask, params):
    p = params
    f32 = jnp.float32
    adl = atom_degree_list.astype(jnp.int32)
    adl_t = jnp.transpose(adl, (0, 2, 1)).reshape(B * K * L)
    am3 = atom_mask[..., None].astype(f32)
    alw = p['align_w_2']
    molw = p['mol_align_w']

    def r2(v):
        return v.reshape(1, -1).astype(f32)

    mol_spec_a = lambda shape: pl.BlockSpec(shape, lambda b: (b, 0, 0))
    par_spec = lambda shape: pl.BlockSpec(shape, lambda b: (0, 0))

    a_ops = [atom_list, p['atom_fc_w'], r2(p['atom_fc_b']),
             alw[:FP], alw[FP:], r2(p['align_b_2']),
             p['attend_w_2'], r2(p['attend_b_2'])]
    a_specs = [mol_spec_a((MPA, L, AF))] + [par_spec(op.shape) for op in a_ops[1:]]
    af, nft, g3, asc3 = pl.pallas_call(
        _stage_a,
        grid=(B // MPA,),
        in_specs=a_specs,
        out_specs=(
            pl.BlockSpec((MPA, L, FP), lambda b: (b, 0, 0)),
            pl.BlockSpec((MPA, FP, L), lambda b: (b, 0, 0)),
            pl.BlockSpec((MPA, L, 1), lambda b: (b, 0, 0)),
            pl.BlockSpec((MPA, L, 1), lambda b: (b, 0, 0)),
        ),
        out_shape=(
            jax.ShapeDtypeStruct((B, L, FP), f32),
            jax.ShapeDtypeStruct((B, FP, L), f32),
            jax.ShapeDtypeStruct((B, L, 1), f32),
            jax.ShapeDtypeStruct((B, L, 1), f32),
        ),
    )(*a_ops)

    ctx_flat = _stage_b(
        nft.reshape(B * L * FP),
        g3.reshape(B * L),
        asc3.reshape(B * L),
        adl_t,
    )
    ctx = ctx_flat.reshape(B, FP, L)

    c_ops = [
        ctx, af, am3,
        p['gru_k_2'], p['gru_rk_2'], r2(p['gru_bi_2']), r2(p['gru_br_2']),
        molw[:FP], molw[FP:], r2(p['mol_align_b']),
        p['mol_attend_w'], r2(p['mol_attend_b']),
        p['mol_gru_k'], p['mol_gru_rk'], r2(p['mol_gru_bi']), r2(p['mol_gru_br']),
        p['lin1_w'], r2(p['lin1_b']), p['lin2_w'], r2(p['lin2_b']),
        p['out_w'], r2(p['out_b']),
    ]
    c_specs = [mol_spec_a((MPC, FP, L)), mol_spec_a((MPC, L, FP)),
               mol_spec_a((MPC, L, 1))]
    c_specs += [par_spec(op.shape) for op in c_ops[3:]]
    out = pl.pallas_call(
        _stage_c,
        grid=(B // MPC,),
        in_specs=c_specs,
        out_specs=pl.BlockSpec((MPC, 1, 1), lambda b: (b, 0, 0)),
        out_shape=jax.ShapeDtypeStruct((B, 1, 1), f32),
    )(*c_ops)
    return out.reshape(B, 1)
